# Initial kernel scaffold; baseline (speedup 1.0000x reference)
import jax
import jax.numpy as jnp
from jax.experimental import pallas as pl


def kernel(x, pos_table):
    raise NotImplementedError("write your pallas kernel here")


# speedup vs baseline: 2.3914x; 2.3914x over previous
import jax
import jax.numpy as jnp
from jax.experimental import pallas as pl

_CHUNK = 512


def _add_kernel(x_ref, pos_ref, out_ref):
    out_ref[...] = x_ref[...] + pos_ref[...][None, :, :]


def kernel(x, pos_table):
    batch, max_len, dim = x.shape
    grid = (max_len // _CHUNK,)
    return pl.pallas_call(
        _add_kernel,
        grid=grid,
        in_specs=[
            pl.BlockSpec((batch, _CHUNK, dim), lambda i: (0, i, 0)),
            pl.BlockSpec((_CHUNK, dim), lambda i: (i, 0)),
        ],
        out_specs=pl.BlockSpec((batch, _CHUNK, dim), lambda i: (0, i, 0)),
        out_shape=jax.ShapeDtypeStruct(x.shape, x.dtype),
    )(x, pos_table)


# speedup vs baseline: 2.4240x; 1.0136x over previous
import jax
import jax.numpy as jnp
from jax.experimental import pallas as pl

_CHUNK = 1024


def _add_kernel(x_ref, pos_ref, out_ref):
    out_ref[...] = x_ref[...] + pos_ref[...][None, :, :]


def kernel(x, pos_table):
    batch, max_len, dim = x.shape
    grid = (max_len // _CHUNK,)
    return pl.pallas_call(
        _add_kernel,
        grid=grid,
        in_specs=[
            pl.BlockSpec((batch, _CHUNK, dim), lambda i: (0, i, 0)),
            pl.BlockSpec((_CHUNK, dim), lambda i: (i, 0)),
        ],
        out_specs=pl.BlockSpec((batch, _CHUNK, dim), lambda i: (0, i, 0)),
        out_shape=jax.ShapeDtypeStruct(x.shape, x.dtype),
    )(x, pos_table)
